# baseline (speedup 1.0000x reference)
import jax
import jax.numpy as jnp
from jax.experimental import pallas as pl

_MASK_PROB = 0.15

_BLK = 1000


def _masks():
    key = jax.random.key(42)
    kn, ke = jax.random.split(key)
    node_mask = jax.random.uniform(kn, (10000,)) < _MASK_PROB
    edge_mask = jax.random.uniform(ke, (320000,)) < _MASK_PROB
    node_keep = 1.0 - node_mask.astype(jnp.float32)
    edge_keep = 1.0 - edge_mask.astype(jnp.float32)
    return node_keep.reshape(10000, 1), edge_keep.reshape(40000, 8)


def _body(nx_blocks, nm_ref, em_ref, x_ref, e_ref, ox_ref, oe_ref):
    i = pl.program_id(0)

    @pl.when(i < nx_blocks)
    def _():
        ox_ref[...] = x_ref[...] * nm_ref[...]

    @pl.when(i >= nx_blocks)
    def _():
        row = jax.lax.broadcasted_iota(jnp.int32, (8, 128), 0)
        lane = jax.lax.broadcasted_iota(jnp.int32, (8, 128), 1)
        expand = (lane // 16 == row).astype(jnp.float32)
        keep = jax.lax.dot(em_ref[...], expand,
                           preferred_element_type=jnp.float32)
        oe_ref[...] = e_ref[...] * keep


def kernel(x, edge_attr):
    n_nodes, dx = x.shape
    n_edges, de = edge_attr.shape
    node_keep, edge_keep = _masks()

    e2 = edge_attr.reshape(n_edges * de // 128, 128)
    n_erows = e2.shape[0]

    nx_blocks = n_nodes // _BLK
    ne_blocks = n_erows // _BLK
    grid = nx_blocks + ne_blocks

    def x_map(i):
        return (jnp.minimum(i, nx_blocks - 1), 0)

    def e_map(i):
        return (jnp.maximum(i - nx_blocks, 0), 0)

    import functools
    body = functools.partial(_body, nx_blocks)

    ox, oe = pl.pallas_call(
        body,
        grid=(grid,),
        in_specs=[
            pl.BlockSpec((_BLK, 1), x_map),
            pl.BlockSpec((_BLK, 8), e_map),
            pl.BlockSpec((_BLK, dx), x_map),
            pl.BlockSpec((_BLK, 128), e_map),
        ],
        out_specs=[
            pl.BlockSpec((_BLK, dx), x_map),
            pl.BlockSpec((_BLK, 128), e_map),
        ],
        out_shape=[
            jax.ShapeDtypeStruct((n_nodes, dx), x.dtype),
            jax.ShapeDtypeStruct((n_erows, 128), edge_attr.dtype),
        ],
    )(node_keep, edge_keep, x, e2)

    return ox, oe.reshape(n_edges, de)
